# baseline (speedup 1.0000x reference)
import functools

import jax
import jax.numpy as jnp
from jax import lax
from jax.experimental import pallas as pl
from jax.experimental.pallas import tpu as pltpu
from jax.experimental.pallas import tpu_sc as plsc

EMBED_DIM = 64


@functools.lru_cache(maxsize=None)
def _make_gather(B, D):
    info = plsc.get_sparse_core_info()
    NC, NS = info.num_cores, info.num_subcores
    NW = NC * NS
    assert B % NW == 0
    b_per_w = B // NW
    CH = 800
    assert b_per_w % CH == 0
    n_ch = b_per_w // CH

    mesh = plsc.VectorSubcoreMesh(core_axis_name="c", subcore_axis_name="s")

    @functools.partial(
        pl.kernel,
        mesh=mesh,
        compiler_params=pltpu.CompilerParams(use_tc_tiling_on_sc=False),
        out_type=jax.ShapeDtypeStruct((B, D), jnp.float32),
        scratch_types=[
            pltpu.VMEM((CH,), jnp.int32),
            pltpu.VMEM((CH, D), jnp.float32),
            pltpu.SemaphoreType.DMA,
        ],
    )
    def gather_kernel(idx_hbm, table_hbm, out_hbm, idx_v, rows_v, sem):
        wid = lax.axis_index("s") * NC + lax.axis_index("c")
        base = wid * b_per_w
        for j in range(n_ch):
            off = base + j * CH
            pltpu.sync_copy(idx_hbm.at[pl.ds(off, CH)], idx_v)
            pltpu.async_copy(table_hbm.at[idx_v], rows_v, sem).wait()
            pltpu.sync_copy(rows_v, out_hbm.at[pl.ds(off, CH)])

    return gather_kernel


def kernel(idx, embeddings):
    n, s = idx.shape
    flat = idx.reshape(n * s).astype(jnp.int32)
    out = _make_gather(n * s, EMBED_DIM)(flat, embeddings)
    return out.reshape(n, s, EMBED_DIM)


# speedup vs baseline: 1.0050x; 1.0050x over previous
import functools

import jax
import jax.numpy as jnp
from jax import lax
from jax.experimental import pallas as pl
from jax.experimental.pallas import tpu as pltpu
from jax.experimental.pallas import tpu_sc as plsc

EMBED_DIM = 64


@functools.lru_cache(maxsize=None)
def _make_gather(B, D):
    info = plsc.get_sparse_core_info()
    NC, NS = info.num_cores, info.num_subcores
    NW = NC * NS
    assert B % NW == 0
    b_per_w = B // NW
    CH = 800
    assert b_per_w % CH == 0
    n_ch = b_per_w // CH

    mesh = plsc.VectorSubcoreMesh(core_axis_name="c", subcore_axis_name="s")

    @functools.partial(
        pl.kernel,
        mesh=mesh,
        compiler_params=pltpu.CompilerParams(use_tc_tiling_on_sc=False),
        out_type=jax.ShapeDtypeStruct((B, D), jnp.float32),
        scratch_types=[
            pltpu.VMEM((CH,), jnp.int32),
            pltpu.VMEM((CH,), jnp.int32),
            pltpu.VMEM((CH, D), jnp.float32),
            pltpu.VMEM((CH, D), jnp.float32),
            pltpu.SemaphoreType.DMA,
            pltpu.SemaphoreType.DMA,
            pltpu.SemaphoreType.DMA,
            pltpu.SemaphoreType.DMA,
            pltpu.SemaphoreType.DMA,
            pltpu.SemaphoreType.DMA,
        ],
    )
    def gather_kernel(idx_hbm, table_hbm, out_hbm,
                      idx_v0, idx_v1, rows_v0, rows_v1,
                      si0, si1, sg0, sg1, so0, so1):
        wid = lax.axis_index("s") * NC + lax.axis_index("c")
        base = wid * b_per_w
        idx_bufs = (idx_v0, idx_v1)
        rows_bufs = (rows_v0, rows_v1)
        si = (si0, si1)
        sg = (sg0, sg1)
        so = (so0, so1)
        copies_i = [None, None]
        copies_o = [None, None]
        copies_i[0] = pltpu.async_copy(
            idx_hbm.at[pl.ds(base, CH)], idx_bufs[0], si[0])
        for j in range(n_ch):
            b = j % 2
            if j + 1 < n_ch:
                nb = (j + 1) % 2
                copies_i[nb] = pltpu.async_copy(
                    idx_hbm.at[pl.ds(base + (j + 1) * CH, CH)],
                    idx_bufs[nb], si[nb])
            copies_i[b].wait()
            if copies_o[b] is not None:
                copies_o[b].wait()
            gather = pltpu.async_copy(
                table_hbm.at[idx_bufs[b]], rows_bufs[b], sg[b])
            gather.wait()
            copies_o[b] = pltpu.async_copy(
                rows_bufs[b], out_hbm.at[pl.ds(base + j * CH, CH)], so[b])
        copies_o[(n_ch - 2) % 2].wait()
        copies_o[(n_ch - 1) % 2].wait()

    return gather_kernel


def kernel(idx, embeddings):
    n, s = idx.shape
    flat = idx.reshape(n * s).astype(jnp.int32)
    out = _make_gather(n * s, EMBED_DIM)(flat, embeddings)
    return out.reshape(n, s, EMBED_DIM)
